# baseline (speedup 1.0000x reference)
import jax
import jax.numpy as jnp
from jax import lax
from jax.experimental import pallas as pl
from jax.experimental.pallas import tpu as pltpu
from jax.experimental.pallas import tpu_sc as plsc

N = 10000
E = 320000
F_IN = 128
D1 = 64
D2 = 16
NEG = 0.2
PW = 128
ACC1 = 72
ACC2 = 32
NWORK = 32
EPW = E // NWORK
K = 40
NCHUNK = EPW // K
NPAD = 10240
TROWS = NPAD // 16
BN = 2000


def _take(v, idx):
    dnums = lax.GatherDimensionNumbers(
        offset_dims=(), collapsed_slice_dims=(0,), start_index_map=(0,))
    return lax.gather(v, idx[:, None], dnums, (1,),
                      mode=lax.GatherScatterMode.PROMISE_IN_BOUNDS)


def _tc1_body(x_ref, w_ref, as_ref, ad_ref, p_ref):
    h = jnp.dot(x_ref[...], w_ref[...], preferred_element_type=jnp.float32)
    asrc = jnp.dot(h, as_ref[...], preferred_element_type=jnp.float32)
    adst = jnp.dot(h, ad_ref[...], preferred_element_type=jnp.float32)
    p_ref[...] = jnp.concatenate(
        [h, asrc, adst, jnp.zeros((BN, PW - D1 - 16), jnp.float32)], axis=1)


_tc1 = pl.pallas_call(
    _tc1_body,
    grid=(N // BN,),
    in_specs=[
        pl.BlockSpec((BN, F_IN), lambda i: (i, 0)),
        pl.BlockSpec((F_IN, D1), lambda i: (0, 0)),
        pl.BlockSpec((D1, 8), lambda i: (0, 0)),
        pl.BlockSpec((D1, 8), lambda i: (0, 0)),
    ],
    out_specs=pl.BlockSpec((BN, PW), lambda i: (i, 0)),
    out_shape=jax.ShapeDtypeStruct((N, PW), jnp.float32),
)


def _tc2_body(acc_ref, w_ref, a_ref, b_ref, p_ref):
    acc = acc_ref[0] + acc_ref[1]
    den = acc[:, 0:8]
    msg = acc[:, 8:ACC1]
    denb = jnp.broadcast_to(den[:, :, None], (BN, 8, 8)).reshape(BN, D1)
    h2 = msg / (denb + 1e-16) + b_ref[...]
    h2 = jnp.where(h2 > 0, h2, jnp.exp(h2) - 1.0)
    z = jnp.dot(h2, w_ref[...], preferred_element_type=jnp.float32)
    asd = jnp.dot(z, a_ref[...], preferred_element_type=jnp.float32)
    p_ref[...] = jnp.concatenate(
        [z, asd, jnp.zeros((BN, PW - D2 - 16), jnp.float32)], axis=1)


_tc2 = pl.pallas_call(
    _tc2_body,
    grid=(N // BN,),
    in_specs=[
        pl.BlockSpec((2, BN, ACC1), lambda i: (0, i, 0)),
        pl.BlockSpec((D1, D2), lambda i: (0, 0)),
        pl.BlockSpec((D2, 16), lambda i: (0, 0)),
        pl.BlockSpec((1, D1), lambda i: (0, 0)),
    ],
    out_specs=pl.BlockSpec((BN, PW), lambda i: (i, 0)),
    out_shape=jax.ShapeDtypeStruct((N, PW), jnp.float32),
)


def _tc3_body(acc_ref, b_ref, out_ref):
    acc = acc_ref[0] + acc_ref[1]
    msg = acc[:, 0:16]
    den = acc[:, 16:17]
    o = msg / (den + 1e-16) + b_ref[...]
    m = jnp.max(o, axis=1, keepdims=True)
    ex = jnp.exp(o - m)
    out_ref[...] = o - m - jnp.log(jnp.sum(ex, axis=1, keepdims=True))


_tc3 = pl.pallas_call(
    _tc3_body,
    grid=(N // BN,),
    in_specs=[
        pl.BlockSpec((2, BN, ACC2), lambda i: (0, i, 0)),
        pl.BlockSpec((1, D2), lambda i: (0, 0)),
    ],
    out_specs=pl.BlockSpec((BN, D2), lambda i: (i, 0)),
    out_shape=jax.ShapeDtypeStruct((N, D2), jnp.float32),
)


_MESH = plsc.VectorSubcoreMesh(core_axis_name="c", subcore_axis_name="s")


def _sc1_body(src_hbm, dst_hbm, p_hbm, zeros_hbm, out_hbm,
              acc_sh, src_va, dst_va, src_vb, dst_vb,
              hg_a, dg_a, hg_b, dg_b, msg_v, sem):
    c = lax.axis_index("c")
    s = lax.axis_index("s")
    w = s * 2 + c
    r0 = s * TROWS
    ebase = w * EPW
    pltpu.sync_copy(zeros_hbm.at[pl.ds(r0, TROWS)], acc_sh.at[pl.ds(r0, TROWS)])
    plsc.subcore_barrier()

    iota = lax.iota(jnp.int32, 16)
    par = iota >> 3

    def idx_descs(i, src_v, dst_v):
        base = ebase + i * K
        return (pltpu.make_async_copy(src_hbm.at[pl.ds(base, K)], src_v, sem),
                pltpu.make_async_copy(dst_hbm.at[pl.ds(base, K)], dst_v, sem))

    def start_idx(i, src_v, dst_v):
        for d in idx_descs(i, src_v, dst_v):
            d.start()

    def wait_idx(i, src_v, dst_v):
        for d in idx_descs(i, src_v, dst_v):
            d.wait()

    def row_descs(src_v, dst_v, hg_v, dg_v):
        return (pltpu.make_async_copy(p_hbm.at[src_v], hg_v, sem),
                pltpu.make_async_copy(p_hbm.at[dst_v], dg_v, sem))

    def start_rows(src_v, dst_v, hg_v, dg_v):
        for d in row_descs(src_v, dst_v, hg_v, dg_v):
            d.start()

    def wait_rows(src_v, dst_v, hg_v, dg_v):
        for d in row_descs(src_v, dst_v, hg_v, dg_v):
            d.wait()

    def work(dst_v, hg_v, dg_v):
        def quad(q, _2):
            for u in range(4):
                k = q * 4 + u
                adst16 = dg_v[k, pl.ds(D1 + 8, 16)]
                asrc16 = hg_v[k, pl.ds(D1, 16)]
                alpha = asrc16 + adst16
                alpha = jnp.where(alpha >= 0, alpha, alpha * NEG)
                e16 = jnp.exp(alpha)
                msg_v[k, pl.ds(0, 16)] = e16
                for v in range(4):
                    att = _take(e16, par + 2 * v)
                    msg_v[k, pl.ds(8 + 16 * v, 16)] = hg_v[k, pl.ds(16 * v, 16)] * att
            return 0

        lax.fori_loop(0, K // 4, quad, 0)
        pltpu.sync_copy(msg_v, acc_sh.at[dst_v], add=True)

    start_idx(0, src_va, dst_va)
    wait_idx(0, src_va, dst_va)
    start_rows(src_va, dst_va, hg_a, dg_a)
    start_idx(1, src_vb, dst_vb)
    last = NCHUNK - 1

    def pair(t, _):
        i = t * 2
        wait_idx(i + 1, src_vb, dst_vb)
        start_rows(src_vb, dst_vb, hg_b, dg_b)
        wait_rows(src_va, dst_va, hg_a, dg_a)
        work(dst_va, hg_a, dg_a)
        i2 = jnp.minimum(i + 2, last)
        start_idx(i2, src_va, dst_va)
        wait_rows(src_vb, dst_vb, hg_b, dg_b)
        work(dst_vb, hg_b, dg_b)
        wait_idx(i2, src_va, dst_va)
        start_rows(src_va, dst_va, hg_a, dg_a)
        start_idx(jnp.minimum(i + 3, last), src_vb, dst_vb)
        return 0

    lax.fori_loop(0, NCHUNK // 2, pair, 0)
    wait_rows(src_va, dst_va, hg_a, dg_a)
    wait_idx(last, src_vb, dst_vb)
    plsc.subcore_barrier()
    pltpu.sync_copy(acc_sh.at[pl.ds(r0, TROWS)], out_hbm.at[c, pl.ds(r0, TROWS)])


_sc1 = pl.kernel(
    _sc1_body,
    out_type=jax.ShapeDtypeStruct((2, NPAD, ACC1), jnp.float32),
    mesh=_MESH,
    scratch_types=[
        pltpu.VMEM_SHARED((NPAD, ACC1), jnp.float32),
        pltpu.VMEM((K,), jnp.int32),
        pltpu.VMEM((K,), jnp.int32),
        pltpu.VMEM((K,), jnp.int32),
        pltpu.VMEM((K,), jnp.int32),
        pltpu.VMEM((K, PW), jnp.float32),
        pltpu.VMEM((K, PW), jnp.float32),
        pltpu.VMEM((K, PW), jnp.float32),
        pltpu.VMEM((K, PW), jnp.float32),
        pltpu.VMEM((K, ACC1), jnp.float32),
        pltpu.SemaphoreType.DMA,
    ],
)


def _sc2_body(src_hbm, dst_hbm, p_hbm, zeros_hbm, out_hbm,
              acc_sh, src_va, dst_va, src_vb, dst_vb,
              zg_a, dg_a, zg_b, dg_b, msg_v, sem):
    c = lax.axis_index("c")
    s = lax.axis_index("s")
    w = s * 2 + c
    r0 = s * TROWS
    ebase = w * EPW
    pltpu.sync_copy(zeros_hbm.at[pl.ds(r0, TROWS)], acc_sh.at[pl.ds(r0, TROWS)])
    plsc.subcore_barrier()

    iota = lax.iota(jnp.int32, 16)
    lane0 = iota == 0
    zero16i = jnp.zeros((16,), jnp.int32)
    one16i = zero16i + 1

    def idx_descs(i, src_v, dst_v):
        base = ebase + i * K
        return (pltpu.make_async_copy(src_hbm.at[pl.ds(base, K)], src_v, sem),
                pltpu.make_async_copy(dst_hbm.at[pl.ds(base, K)], dst_v, sem))

    def start_idx(i, src_v, dst_v):
        for d in idx_descs(i, src_v, dst_v):
            d.start()

    def wait_idx(i, src_v, dst_v):
        for d in idx_descs(i, src_v, dst_v):
            d.wait()

    def row_descs(src_v, dst_v, zg_v, dg_v):
        return (pltpu.make_async_copy(p_hbm.at[src_v], zg_v, sem),
                pltpu.make_async_copy(p_hbm.at[dst_v], dg_v, sem))

    def start_rows(src_v, dst_v, zg_v, dg_v):
        for d in row_descs(src_v, dst_v, zg_v, dg_v):
            d.start()

    def wait_rows(src_v, dst_v, zg_v, dg_v):
        for d in row_descs(src_v, dst_v, zg_v, dg_v):
            d.wait()

    def work(dst_v, zg_v, dg_v):
        def quad(q, _2):
            for u in range(4):
                k = q * 4 + u
                a_s = _take(zg_v[k, pl.ds(D2, 16)], zero16i)
                a_d = _take(dg_v[k, pl.ds(D2, 16)], one16i)
                alpha = a_s + a_d
                alpha = jnp.where(alpha >= 0, alpha, alpha * NEG)
                ev = jnp.exp(alpha)
                msg_v[k, pl.ds(0, 16)] = zg_v[k, pl.ds(0, 16)] * ev
                msg_v[k, pl.ds(16, 16)] = jnp.where(lane0, ev, 0.0)
            return 0

        lax.fori_loop(0, K // 4, quad, 0)
        pltpu.sync_copy(msg_v, acc_sh.at[dst_v], add=True)

    start_idx(0, src_va, dst_va)
    wait_idx(0, src_va, dst_va)
    start_rows(src_va, dst_va, zg_a, dg_a)
    start_idx(1, src_vb, dst_vb)
    last = NCHUNK - 1

    def pair(t, _):
        i = t * 2
        wait_idx(i + 1, src_vb, dst_vb)
        start_rows(src_vb, dst_vb, zg_b, dg_b)
        wait_rows(src_va, dst_va, zg_a, dg_a)
        work(dst_va, zg_a, dg_a)
        i2 = jnp.minimum(i + 2, last)
        start_idx(i2, src_va, dst_va)
        wait_rows(src_vb, dst_vb, zg_b, dg_b)
        work(dst_vb, zg_b, dg_b)
        wait_idx(i2, src_va, dst_va)
        start_rows(src_va, dst_va, zg_a, dg_a)
        start_idx(jnp.minimum(i + 3, last), src_vb, dst_vb)
        return 0

    lax.fori_loop(0, NCHUNK // 2, pair, 0)
    wait_rows(src_va, dst_va, zg_a, dg_a)
    wait_idx(last, src_vb, dst_vb)
    plsc.subcore_barrier()
    pltpu.sync_copy(acc_sh.at[pl.ds(r0, TROWS)], out_hbm.at[c, pl.ds(r0, TROWS)])


_sc2 = pl.kernel(
    _sc2_body,
    out_type=jax.ShapeDtypeStruct((2, NPAD, ACC2), jnp.float32),
    mesh=_MESH,
    scratch_types=[
        pltpu.VMEM_SHARED((NPAD, ACC2), jnp.float32),
        pltpu.VMEM((K,), jnp.int32),
        pltpu.VMEM((K,), jnp.int32),
        pltpu.VMEM((K,), jnp.int32),
        pltpu.VMEM((K,), jnp.int32),
        pltpu.VMEM((K, PW), jnp.float32),
        pltpu.VMEM((K, PW), jnp.float32),
        pltpu.VMEM((K, PW), jnp.float32),
        pltpu.VMEM((K, PW), jnp.float32),
        pltpu.VMEM((K, ACC2), jnp.float32),
        pltpu.SemaphoreType.DMA,
    ],
)


def kernel(x, edge_index, W1, att_src1, att_dst1, bias1,
           W2, att_src2, att_dst2, bias2):
    f32 = jnp.float32
    eye8 = jnp.eye(8, dtype=f32)
    a_src_m = (att_src1[:, :, None] * eye8[:, None, :]).reshape(D1, 8)
    a_dst_m = (att_dst1[:, :, None] * eye8[:, None, :]).reshape(D1, 8)
    a2 = jnp.concatenate(
        [att_src2.T, att_dst2.T, jnp.zeros((D2, 14), f32)], axis=1)

    p1 = _tc1(x, W1, a_src_m, a_dst_m)
    srcs = edge_index[0]
    dsts = edge_index[1]
    acc1 = _sc1(srcs, dsts, p1, jnp.zeros((NPAD, ACC1), f32))
    p2 = _tc2(acc1, W2, a2, bias1.reshape(1, D1))
    acc2 = _sc2(srcs, dsts, p2, jnp.zeros((NPAD, ACC2), f32))
    return _tc3(acc2, bias2.reshape(1, D2))


# speedup vs baseline: 1.1529x; 1.1529x over previous
import jax
import jax.numpy as jnp
from jax import lax
from jax.experimental import pallas as pl
from jax.experimental.pallas import tpu as pltpu
from jax.experimental.pallas import tpu_sc as plsc

N = 10000
E = 320000
F_IN = 128
D1 = 64
D2 = 16
NEG = 0.2
PW = 128
ACC1 = 72
ACC2 = 32
NWORK = 32
EPW = E // NWORK
K = 40
NCHUNK = EPW // K
NPAD = 10240
TROWS = NPAD // 16
BN = 2000


def _take(v, idx):
    dnums = lax.GatherDimensionNumbers(
        offset_dims=(), collapsed_slice_dims=(0,), start_index_map=(0,))
    return lax.gather(v, idx[:, None], dnums, (1,),
                      mode=lax.GatherScatterMode.PROMISE_IN_BOUNDS)


def _tc1_body(x_ref, w_ref, as_ref, ad_ref, p_ref):
    h = jnp.dot(x_ref[...], w_ref[...], preferred_element_type=jnp.float32)
    asrc = jnp.dot(h, as_ref[...], preferred_element_type=jnp.float32)
    adst = jnp.dot(h, ad_ref[...], preferred_element_type=jnp.float32)
    p_ref[...] = jnp.concatenate(
        [h, asrc, adst, jnp.zeros((BN, PW - D1 - 16), jnp.float32)], axis=1)


_tc1 = pl.pallas_call(
    _tc1_body,
    grid=(N // BN,),
    in_specs=[
        pl.BlockSpec((BN, F_IN), lambda i: (i, 0)),
        pl.BlockSpec((F_IN, D1), lambda i: (0, 0)),
        pl.BlockSpec((D1, 8), lambda i: (0, 0)),
        pl.BlockSpec((D1, 8), lambda i: (0, 0)),
    ],
    out_specs=pl.BlockSpec((BN, PW), lambda i: (i, 0)),
    out_shape=jax.ShapeDtypeStruct((N, PW), jnp.float32),
)


def _tc2_body(acc_ref, w_ref, a_ref, b_ref, p_ref):
    acc = acc_ref[0] + acc_ref[1]
    den = acc[:, 0:8]
    msg = acc[:, 8:ACC1]
    denb = jnp.broadcast_to(den[:, :, None], (BN, 8, 8)).reshape(BN, D1)
    h2 = msg / (denb + 1e-16) + b_ref[...]
    h2 = jnp.where(h2 > 0, h2, jnp.exp(h2) - 1.0)
    z = jnp.dot(h2, w_ref[...], preferred_element_type=jnp.float32)
    asd = jnp.dot(z, a_ref[...], preferred_element_type=jnp.float32)
    p_ref[...] = jnp.concatenate(
        [z, asd, jnp.zeros((BN, PW - D2 - 16), jnp.float32)], axis=1)


_tc2 = pl.pallas_call(
    _tc2_body,
    grid=(N // BN,),
    in_specs=[
        pl.BlockSpec((2, BN, ACC1), lambda i: (0, i, 0)),
        pl.BlockSpec((D1, D2), lambda i: (0, 0)),
        pl.BlockSpec((D2, 16), lambda i: (0, 0)),
        pl.BlockSpec((1, D1), lambda i: (0, 0)),
    ],
    out_specs=pl.BlockSpec((BN, PW), lambda i: (i, 0)),
    out_shape=jax.ShapeDtypeStruct((N, PW), jnp.float32),
)


def _tc3_body(acc_ref, b_ref, out_ref):
    acc = acc_ref[0] + acc_ref[1]
    msg = acc[:, 0:16]
    den = acc[:, 16:17]
    o = msg / (den + 1e-16) + b_ref[...]
    m = jnp.max(o, axis=1, keepdims=True)
    ex = jnp.exp(o - m)
    out_ref[...] = o - m - jnp.log(jnp.sum(ex, axis=1, keepdims=True))


_tc3 = pl.pallas_call(
    _tc3_body,
    grid=(N // BN,),
    in_specs=[
        pl.BlockSpec((2, BN, ACC2), lambda i: (0, i, 0)),
        pl.BlockSpec((1, D2), lambda i: (0, 0)),
    ],
    out_specs=pl.BlockSpec((BN, D2), lambda i: (i, 0)),
    out_shape=jax.ShapeDtypeStruct((N, D2), jnp.float32),
)


_MESH = plsc.VectorSubcoreMesh(core_axis_name="c", subcore_axis_name="s")


def _sc1_body(src_hbm, dst_hbm, p_hbm, zeros_hbm, out_hbm,
              acc_sh, src_va, dst_va, src_vb, dst_vb,
              hg_a, dg_a, hg_b, dg_b, msg_v, sem):
    c = lax.axis_index("c")
    s = lax.axis_index("s")
    w = s * 2 + c
    r0 = s * TROWS
    ebase = w * EPW
    pltpu.sync_copy(zeros_hbm.at[pl.ds(r0, TROWS)], acc_sh.at[pl.ds(r0, TROWS)])
    plsc.subcore_barrier()

    iota = lax.iota(jnp.int32, 16)
    par = iota >> 3

    def fetch(i, src_v, dst_v, hg_v, dg_v):
        base = ebase + i * K
        pltpu.sync_copy(src_hbm.at[pl.ds(base, K)], src_v)
        pltpu.sync_copy(dst_hbm.at[pl.ds(base, K)], dst_v)
        pltpu.async_copy(p_hbm.at[src_v], hg_v, sem)
        pltpu.async_copy(p_hbm.at[dst_v], dg_v, sem)

    def drain(src_v, dst_v, hg_v, dg_v):
        pltpu.make_async_copy(p_hbm.at[src_v], hg_v, sem).wait()
        pltpu.make_async_copy(p_hbm.at[dst_v], dg_v, sem).wait()

    def work(dst_v, hg_v, dg_v):
        def quad(q, _2):
            for u in range(4):
                k = q * 4 + u
                adst16 = dg_v[k, pl.ds(D1 + 8, 16)]
                asrc16 = hg_v[k, pl.ds(D1, 16)]
                alpha = asrc16 + adst16
                alpha = jnp.where(alpha >= 0, alpha, alpha * NEG)
                e16 = jnp.exp(alpha)
                msg_v[k, pl.ds(0, 16)] = e16
                for v in range(4):
                    att = _take(e16, par + 2 * v)
                    msg_v[k, pl.ds(8 + 16 * v, 16)] = hg_v[k, pl.ds(16 * v, 16)] * att
            return 0

        lax.fori_loop(0, K // 4, quad, 0)
        pltpu.sync_copy(msg_v, acc_sh.at[dst_v], add=True)

    fetch(0, src_va, dst_va, hg_a, dg_a)

    def pair(t, _):
        i = t * 2
        fetch(i + 1, src_vb, dst_vb, hg_b, dg_b)
        drain(src_va, dst_va, hg_a, dg_a)
        work(dst_va, hg_a, dg_a)
        fetch(jnp.minimum(i + 2, NCHUNK - 1), src_va, dst_va, hg_a, dg_a)
        drain(src_vb, dst_vb, hg_b, dg_b)
        work(dst_vb, hg_b, dg_b)
        return 0

    lax.fori_loop(0, NCHUNK // 2, pair, 0)
    drain(src_va, dst_va, hg_a, dg_a)
    plsc.subcore_barrier()
    pltpu.sync_copy(acc_sh.at[pl.ds(r0, TROWS)], out_hbm.at[c, pl.ds(r0, TROWS)])


_sc1 = pl.kernel(
    _sc1_body,
    out_type=jax.ShapeDtypeStruct((2, NPAD, ACC1), jnp.float32),
    mesh=_MESH,
    scratch_types=[
        pltpu.VMEM_SHARED((NPAD, ACC1), jnp.float32),
        pltpu.VMEM((K,), jnp.int32),
        pltpu.VMEM((K,), jnp.int32),
        pltpu.VMEM((K,), jnp.int32),
        pltpu.VMEM((K,), jnp.int32),
        pltpu.VMEM((K, PW), jnp.float32),
        pltpu.VMEM((K, PW), jnp.float32),
        pltpu.VMEM((K, PW), jnp.float32),
        pltpu.VMEM((K, PW), jnp.float32),
        pltpu.VMEM((K, ACC1), jnp.float32),
        pltpu.SemaphoreType.DMA,
    ],
)


def _sc2_body(src_hbm, dst_hbm, p_hbm, zeros_hbm, out_hbm,
              acc_sh, src_va, dst_va, src_vb, dst_vb,
              zg_a, dg_a, zg_b, dg_b, msg_v, sem):
    c = lax.axis_index("c")
    s = lax.axis_index("s")
    w = s * 2 + c
    r0 = s * TROWS
    ebase = w * EPW
    pltpu.sync_copy(zeros_hbm.at[pl.ds(r0, TROWS)], acc_sh.at[pl.ds(r0, TROWS)])
    plsc.subcore_barrier()

    iota = lax.iota(jnp.int32, 16)
    lane0 = iota == 0
    zero16i = jnp.zeros((16,), jnp.int32)
    one16i = zero16i + 1

    def fetch(i, src_v, dst_v, zg_v, dg_v):
        base = ebase + i * K
        pltpu.sync_copy(src_hbm.at[pl.ds(base, K)], src_v)
        pltpu.sync_copy(dst_hbm.at[pl.ds(base, K)], dst_v)
        pltpu.async_copy(p_hbm.at[src_v], zg_v, sem)
        pltpu.async_copy(p_hbm.at[dst_v], dg_v, sem)

    def drain(src_v, dst_v, zg_v, dg_v):
        pltpu.make_async_copy(p_hbm.at[src_v], zg_v, sem).wait()
        pltpu.make_async_copy(p_hbm.at[dst_v], dg_v, sem).wait()

    def work(dst_v, zg_v, dg_v):
        def quad(q, _2):
            for u in range(4):
                k = q * 4 + u
                a_s = _take(zg_v[k, pl.ds(D2, 16)], zero16i)
                a_d = _take(dg_v[k, pl.ds(D2, 16)], one16i)
                alpha = a_s + a_d
                alpha = jnp.where(alpha >= 0, alpha, alpha * NEG)
                ev = jnp.exp(alpha)
                msg_v[k, pl.ds(0, 16)] = zg_v[k, pl.ds(0, 16)] * ev
                msg_v[k, pl.ds(16, 16)] = jnp.where(lane0, ev, 0.0)
            return 0

        lax.fori_loop(0, K // 4, quad, 0)
        pltpu.sync_copy(msg_v, acc_sh.at[dst_v], add=True)

    fetch(0, src_va, dst_va, zg_a, dg_a)

    def pair(t, _):
        i = t * 2
        fetch(i + 1, src_vb, dst_vb, zg_b, dg_b)
        drain(src_va, dst_va, zg_a, dg_a)
        work(dst_va, zg_a, dg_a)
        fetch(jnp.minimum(i + 2, NCHUNK - 1), src_va, dst_va, zg_a, dg_a)
        drain(src_vb, dst_vb, zg_b, dg_b)
        work(dst_vb, zg_b, dg_b)
        return 0

    lax.fori_loop(0, NCHUNK // 2, pair, 0)
    drain(src_va, dst_va, zg_a, dg_a)
    plsc.subcore_barrier()
    pltpu.sync_copy(acc_sh.at[pl.ds(r0, TROWS)], out_hbm.at[c, pl.ds(r0, TROWS)])


_sc2 = pl.kernel(
    _sc2_body,
    out_type=jax.ShapeDtypeStruct((2, NPAD, ACC2), jnp.float32),
    mesh=_MESH,
    scratch_types=[
        pltpu.VMEM_SHARED((NPAD, ACC2), jnp.float32),
        pltpu.VMEM((K,), jnp.int32),
        pltpu.VMEM((K,), jnp.int32),
        pltpu.VMEM((K,), jnp.int32),
        pltpu.VMEM((K,), jnp.int32),
        pltpu.VMEM((K, PW), jnp.float32),
        pltpu.VMEM((K, PW), jnp.float32),
        pltpu.VMEM((K, PW), jnp.float32),
        pltpu.VMEM((K, PW), jnp.float32),
        pltpu.VMEM((K, ACC2), jnp.float32),
        pltpu.SemaphoreType.DMA,
    ],
)


def kernel(x, edge_index, W1, att_src1, att_dst1, bias1,
           W2, att_src2, att_dst2, bias2):
    f32 = jnp.float32
    eye8 = jnp.eye(8, dtype=f32)
    a_src_m = (att_src1[:, :, None] * eye8[:, None, :]).reshape(D1, 8)
    a_dst_m = (att_dst1[:, :, None] * eye8[:, None, :]).reshape(D1, 8)
    a2 = jnp.concatenate(
        [att_src2.T, att_dst2.T, jnp.zeros((D2, 14), f32)], axis=1)

    p1 = _tc1(x, W1, a_src_m, a_dst_m)
    srcs = edge_index[0]
    dsts = edge_index[1]
    acc1 = _sc1(srcs, dsts, p1, jnp.zeros((NPAD, ACC1), f32))
    p2 = _tc2(acc1, W2, a2, bias1.reshape(1, D1))
    acc2 = _sc2(srcs, dsts, p2, jnp.zeros((NPAD, ACC2), f32))
    return _tc3(acc2, bias2.reshape(1, D2))
